# baseline (speedup 1.0000x reference)
import functools

import jax
import jax.numpy as jnp
from jax import lax
from jax.experimental import pallas as pl
from jax.experimental.pallas import tpu as pltpu
from jax.experimental.pallas import tpu_sc as plsc

N = 10000
E = 160000
IN_DIM = 128
HID = 256
OUT_DIM = 32

NP = 10240
EP = 163840
NC = 2
NS = 16
L = 16
CH = 128
ROWS_B1 = EP // (NC * NS) // CH
ROWS_AGG = EP // NS // CH
NPT = NP // NS

_f32 = jnp.float32
_i32 = jnp.int32


def _elu(x):
    return jnp.where(x > 0, x, jnp.exp(jnp.minimum(x, 0.0)) - 1.0)


def _tc1_body(feat_ref, w1_ref, atts_ref, attd_ref, as_ref, ad_ref):
    xb = jnp.dot(feat_ref[...], w1_ref[...], preferred_element_type=_f32)
    as_ref[...] = jnp.sum(xb * atts_ref[...], axis=-1, keepdims=True)
    ad_ref[...] = jnp.sum(xb * attd_ref[...], axis=-1, keepdims=True)


def _tc1(featp, w1, atts2, attd2):
    R = 512
    grid = (NP // R,)
    return pl.pallas_call(
        _tc1_body,
        grid=grid,
        in_specs=[
            pl.BlockSpec((R, IN_DIM), lambda i: (i, 0)),
            pl.BlockSpec((IN_DIM, HID), lambda i: (0, 0)),
            pl.BlockSpec((1, HID), lambda i: (0, 0)),
            pl.BlockSpec((1, HID), lambda i: (0, 0)),
        ],
        out_specs=[
            pl.BlockSpec((R, 1), lambda i: (i, 0)),
            pl.BlockSpec((R, 1), lambda i: (i, 0)),
        ],
        out_shape=[
            jax.ShapeDtypeStruct((NP, 1), _f32),
            jax.ShapeDtypeStruct((NP, 1), _f32),
        ],
    )(featp, w1, atts2, attd2)


def _tc2_body(dp_ref, out_ref):
    out_ref[...] = 1.0 / (dp_ref[0] + dp_ref[1] + 1e-16)


def _tc2(denom_parts3):
    return pl.pallas_call(
        _tc2_body,
        out_shape=jax.ShapeDtypeStruct((NP // 128, 128), _f32),
    )(denom_parts3)


def _tc3_body(af_ref, w1_ref, w2_ref, h2_ref):
    h1 = _elu(jnp.dot(af_ref[...], w1_ref[...], preferred_element_type=_f32))
    h2_ref[...] = jnp.dot(h1, w2_ref[...], preferred_element_type=_f32)


def _tc3(af, w1, w2):
    R = 512
    grid = (NP // R,)
    return pl.pallas_call(
        _tc3_body,
        grid=grid,
        in_specs=[
            pl.BlockSpec((R, IN_DIM), lambda i: (i, 0)),
            pl.BlockSpec((IN_DIM, HID), lambda i: (0, 0)),
            pl.BlockSpec((HID, OUT_DIM), lambda i: (0, 0)),
        ],
        out_specs=pl.BlockSpec((R, OUT_DIM), lambda i: (i, 0)),
        out_shape=jax.ShapeDtypeStruct((NP, OUT_DIM), _f32),
    )(af, w1, w2)


def _tc4_body(ah_ref, w2_ref, w1_ref, h4_ref, pi_ref, disp_ref, mean_ref):
    ah = ah_ref[0] + ah_ref[1]
    t = lax.dot_general(ah, w2_ref[...], (((1,), (1,)), ((), ())),
                        preferred_element_type=_f32)
    h3 = _elu(t)
    h4 = lax.dot_general(h3, w1_ref[...], (((1,), (1,)), ((), ())),
                         preferred_element_type=_f32)
    h4_ref[...] = h4
    pi_ref[...] = 1.0 / (1.0 + jnp.exp(-h4))
    sp = jnp.log(1.0 + jnp.exp(-jnp.abs(h4))) + jnp.maximum(h4, 0.0)
    disp_ref[...] = jnp.clip(sp, 0.0001, 10000.0)
    mean_ref[...] = jnp.clip(jnp.exp(h4), 1e-05, 1000000.0)


def _tc4(ah_parts, w2, w1):
    R = 1000
    grid = (N // R,)
    o = jax.ShapeDtypeStruct((N, IN_DIM), _f32)
    return pl.pallas_call(
        _tc4_body,
        grid=grid,
        in_specs=[
            pl.BlockSpec((NC, R, OUT_DIM), lambda i: (0, i, 0)),
            pl.BlockSpec((HID, OUT_DIM), lambda i: (0, 0)),
            pl.BlockSpec((IN_DIM, HID), lambda i: (0, 0)),
        ],
        out_specs=[pl.BlockSpec((R, IN_DIM), lambda i: (i, 0))] * 4,
        out_shape=[o, o, o, o],
    )(ah_parts, w2, w1)


@functools.lru_cache(maxsize=None)
def _mesh():
    return plsc.VectorSubcoreMesh(core_axis_name="c", subcore_axis_name="s",
                                  num_cores=NC, num_subcores=NS)


def _zero_vmem(ref, nrows):
    z = jnp.zeros((L,), _f32)

    def body(j, _):
        for q in range(CH // L):
            ref[j, pl.ds(q * L, L)] = z
        return 0

    lax.fori_loop(0, nrows, body, 0)


def _b1_body(src_hbm, dst_hbm, asrc_hbm, adst_hbm, ex_hbm, dpart_hbm,
             asv, adv, srcv, dstv, exv, zbuf, denom_sh):
    c = lax.axis_index("c")
    s = lax.axis_index("s")
    wid = s * NC + c

    pltpu.sync_copy(asrc_hbm, asv)
    pltpu.sync_copy(adst_hbm, adv)
    pltpu.sync_copy(src_hbm.at[pl.ds(wid * ROWS_B1, ROWS_B1)], srcv)
    pltpu.sync_copy(dst_hbm.at[pl.ds(wid * ROWS_B1, ROWS_B1)], dstv)

    z = jnp.zeros((L,), _f32)
    def zb(j, _):
        zbuf[pl.ds(j * L, L)] = z
        return 0
    lax.fori_loop(0, NPT // L, zb, 0)
    pltpu.sync_copy(zbuf, denom_sh.at[pl.ds(s * NPT, NPT)])
    plsc.subcore_barrier()

    def chunk(j, _):
        for q in range(CH // L):
            sv = srcv[j, pl.ds(q * L, L)]
            dv = dstv[j, pl.ds(q * L, L)]
            a1 = plsc.load_gather(asv, [sv])
            a2 = plsc.load_gather(adv, [dv])
            x = a1 + a2
            sig = 1.0 / (1.0 + jnp.exp(-x))
            exv[j, pl.ds(q * L, L)] = jnp.exp(sig)
        return 0

    lax.fori_loop(0, ROWS_B1, chunk, 0)

    def scat(j, _):
        pltpu.sync_copy(exv.at[j], denom_sh.at[dstv.at[j]], add=True)
        return 0

    lax.fori_loop(0, ROWS_B1, scat, 0)
    pltpu.sync_copy(exv, ex_hbm.at[pl.ds(wid * ROWS_B1, ROWS_B1)])
    plsc.subcore_barrier()

    pltpu.sync_copy(denom_sh.at[pl.ds(s * NPT, NPT)],
                    dpart_hbm.at[pl.ds(c * NP + s * NPT, NPT)])


@functools.lru_cache(maxsize=None)
def _b1_kernel():
    return pl.kernel(
        _b1_body,
        out_type=[
            jax.ShapeDtypeStruct((EP // CH, CH), _f32),
            jax.ShapeDtypeStruct((NC * NP,), _f32),
        ],
        mesh=_mesh(),
        compiler_params=pltpu.CompilerParams(use_tc_tiling_on_sc=False, needs_layout_passes=False),
        scratch_types=[
            pltpu.VMEM((NP,), _f32),
            pltpu.VMEM((NP,), _f32),
            pltpu.VMEM((ROWS_B1, CH), _i32),
            pltpu.VMEM((ROWS_B1, CH), _i32),
            pltpu.VMEM((ROWS_B1, CH), _f32),
            pltpu.VMEM((NPT,), _f32),
            pltpu.VMEM_SHARED((NP,), _f32),
        ],
    )


G_AGG = 10


def _make_agg_body(d, chunks_per_tile, dim_split):
    n_grp = chunks_per_tile // G_AGG
    qd = d // L

    def body(src_hbm, dst_hbm, ex_hbm, dpart_hbm, table_hbm, out_hbm,
             srcv, dstv, exv, invdv, tmpv, wv, rows0, rows1,
             gsem0, gsem1, ssem0, ssem1, accum_sh):
        c = lax.axis_index("c")
        s = lax.axis_index("s")

        pltpu.sync_copy(dpart_hbm.at[pl.ds(0, NP)], invdv)
        for k in range(NP // NPT):
            pltpu.sync_copy(dpart_hbm.at[pl.ds(NP + k * NPT, NPT)], tmpv)

            def inv(j, _, _k=k):
                b = _k * NPT + j * L
                invdv[pl.ds(b, L)] = 1.0 / (
                    invdv[pl.ds(b, L)] + tmpv[pl.ds(j * L, L)] + 1e-16)
                return 0
            lax.fori_loop(0, NPT // L, inv, 0)

        if dim_split:
            off = (c * NP).astype(_i32)
            tile_chunk0 = s * chunks_per_tile
        else:
            off = None
            tile_chunk0 = (c * NS + s) * chunks_per_tile

        z = jnp.zeros((L,), _f32)

        def zrow(e, _):
            for q in range(qd):
                rows0[e, pl.ds(q * L, L)] = z
            return 0
        lax.fori_loop(0, CH, zrow, 0)
        for r0 in range(0, NPT, CH):
            pltpu.sync_copy(rows0, accum_sh.at[pl.ds(s * NPT + r0, CH)])
        plsc.subcore_barrier()

        bufs = ((rows0, gsem0, ssem0), (rows1, gsem1, ssem1))

        def group(g, _):
            base = tile_chunk0 + g * G_AGG
            pltpu.sync_copy(src_hbm.at[pl.ds(base, G_AGG)], srcv)
            pltpu.sync_copy(dst_hbm.at[pl.ds(base, G_AGG)], dstv)
            pltpu.sync_copy(ex_hbm.at[pl.ds(base, G_AGG)], exv)

            for j in range(G_AGG):
                for q in range(CH // L):
                    dv = dstv[j, pl.ds(q * L, L)]
                    iv = plsc.load_gather(invdv, [dv])
                    wv[j, pl.ds(q * L, L)] = exv[j, pl.ds(q * L, L)] * iv
                    if off is not None:
                        srcv[j, pl.ds(q * L, L)] = (
                            srcv[j, pl.ds(q * L, L)] + off)

            gd = [None] * G_AGG
            sd = [None] * G_AGG
            gd[0] = pltpu.async_copy(table_hbm.at[srcv.at[0]], rows0, gsem0)
            for j in range(G_AGG):
                rb, _, ssb = bufs[j & 1]
                ro, gso, _ = bufs[(j + 1) & 1]
                if j + 1 < G_AGG:
                    if j >= 1:
                        sd[j - 1].wait()
                    gd[j + 1] = pltpu.async_copy(
                        table_hbm.at[srcv.at[j + 1]], ro, gso)
                gd[j].wait()

                @pl.loop(0, CH, unroll=4)
                def _scale(e):
                    wbc = plsc.load_gather(
                        wv, [jnp.full((L,), j, _i32), jnp.full((L,), e, _i32)])
                    for q in range(qd):
                        rb[e, pl.ds(q * L, L)] = rb[e, pl.ds(q * L, L)] * wbc

                sd[j] = pltpu.async_copy(rb, accum_sh.at[dstv.at[j]], ssb,
                                         add=True)
            sd[G_AGG - 2].wait()
            sd[G_AGG - 1].wait()
            return 0

        lax.fori_loop(0, n_grp, group, 0)
        plsc.subcore_barrier()
        pltpu.sync_copy(accum_sh.at[pl.ds(s * NPT, NPT)],
                        out_hbm.at[pl.ds(c * NP + s * NPT, NPT)])

    return body


@functools.lru_cache(maxsize=None)
def _agg_kernel(d, chunks_per_tile, dim_split):
    return pl.kernel(
        _make_agg_body(d, chunks_per_tile, dim_split),
        out_type=jax.ShapeDtypeStruct((NC * NP, d), _f32),
        mesh=_mesh(),
        compiler_params=pltpu.CompilerParams(use_tc_tiling_on_sc=False, needs_layout_passes=False),
        scratch_types=[
            pltpu.VMEM((G_AGG, CH), _i32),
            pltpu.VMEM((G_AGG, CH), _i32),
            pltpu.VMEM((G_AGG, CH), _f32),
            pltpu.VMEM((NP,), _f32),
            pltpu.VMEM((NPT,), _f32),
            pltpu.VMEM((G_AGG, CH), _f32),
            pltpu.VMEM((CH, d), _f32),
            pltpu.VMEM((CH, d), _f32),
            pltpu.SemaphoreType.DMA,
            pltpu.SemaphoreType.DMA,
            pltpu.SemaphoreType.DMA,
            pltpu.SemaphoreType.DMA,
            pltpu.VMEM_SHARED((NP, d), _f32),
        ],
    )


def _csl_body(gat_hbm, sct_hbm, h2_hbm, out_hbm,
              gv, sv, onesv, rows0, rows1, cntv, mv,
              gsem0, gsem1, ssem0, ssem1, csem, accum_sh, cnt_sh):
    c = lax.axis_index("c")
    s = lax.axis_index("s")
    base = c * (EP // CH) + s * ROWS_CSL

    pltpu.sync_copy(gat_hbm.at[pl.ds(base, ROWS_CSL)], gv)
    pltpu.sync_copy(sct_hbm.at[pl.ds(base, ROWS_CSL)], sv)

    one = jnp.full((L,), 1.0, _f32)
    z = jnp.zeros((L,), _f32)
    for q in range(CH // L):
        onesv[pl.ds(q * L, L)] = one

    def zrow(e, _):
        for q in range(OUT_DIM // L):
            rows0[e, pl.ds(q * L, L)] = z
        return 0
    lax.fori_loop(0, CH, zrow, 0)

    for r0 in range(0, NPT, CH):
        pltpu.sync_copy(rows0, accum_sh.at[pl.ds(s * NPT + r0, CH)])
    def zb(k, _):
        cntv[pl.ds(k * L, L)] = z
        return 0
    lax.fori_loop(0, NPT // L, zb, 0)
    pltpu.sync_copy(cntv, cnt_sh.at[pl.ds(s * NPT, NPT)])
    plsc.subcore_barrier()

    bufs = ((rows0, gsem0, ssem0), (rows1, gsem1, ssem1))
    gd = [None] * ROWS_CSL
    sd = [None] * ROWS_CSL
    cd = [None] * ROWS_CSL
    gd[0] = pltpu.async_copy(h2_hbm.at[gv.at[0]], rows0, gsem0)
    for j in range(ROWS_CSL):
        rb, _, ssb = bufs[j & 1]
        ro, gso, _ = bufs[(j + 1) & 1]
        if j + 1 < ROWS_CSL:
            if j >= 1:
                sd[j - 1].wait()
            gd[j + 1] = pltpu.async_copy(h2_hbm.at[gv.at[j + 1]], ro, gso)
        gd[j].wait()
        sd[j] = pltpu.async_copy(rb, accum_sh.at[sv.at[j]], ssb, add=True)
        cd[j] = pltpu.async_copy(onesv, cnt_sh.at[sv.at[j]], csem, add=True)
        if j >= 1:
            cd[j - 1].wait()
    sd[ROWS_CSL - 2].wait()
    sd[ROWS_CSL - 1].wait()
    cd[ROWS_CSL - 1].wait()
    plsc.subcore_barrier()

    pltpu.sync_copy(cnt_sh.at[pl.ds(s * NPT, NPT)], cntv)
    pltpu.sync_copy(accum_sh.at[pl.ds(s * NPT, NPT)], mv)

    def div(r, _):
        cb = plsc.load_gather(cntv, [jnp.full((L,), r, _i32)])
        rec = 1.0 / jnp.maximum(cb, 1.0)
        for q in range(OUT_DIM // L):
            mv[r, pl.ds(q * L, L)] = mv[r, pl.ds(q * L, L)] * rec
        return 0

    lax.fori_loop(0, NPT, div, 0)
    pltpu.sync_copy(mv, out_hbm.at[pl.ds(c * NP + s * NPT, NPT)])


ROWS_CSL = EP // NS // CH


@functools.lru_cache(maxsize=None)
def _csl_kernel():
    return pl.kernel(
        _csl_body,
        out_type=jax.ShapeDtypeStruct((NC * NP, OUT_DIM), _f32),
        mesh=_mesh(),
        compiler_params=pltpu.CompilerParams(use_tc_tiling_on_sc=False, needs_layout_passes=False),
        scratch_types=[
            pltpu.VMEM((ROWS_CSL, CH), _i32),
            pltpu.VMEM((ROWS_CSL, CH), _i32),
            pltpu.VMEM((CH,), _f32),
            pltpu.VMEM((CH, OUT_DIM), _f32),
            pltpu.VMEM((CH, OUT_DIM), _f32),
            pltpu.VMEM((NPT,), _f32),
            pltpu.VMEM((NPT, OUT_DIM), _f32),
            pltpu.SemaphoreType.DMA,
            pltpu.SemaphoreType.DMA,
            pltpu.SemaphoreType.DMA,
            pltpu.SemaphoreType.DMA,
            pltpu.SemaphoreType.DMA,
            pltpu.VMEM_SHARED((NP, OUT_DIM), _f32),
            pltpu.VMEM_SHARED((NP,), _f32),
        ],
    )


def _pad_edges(idx):
    return jnp.pad(idx, (0, EP - E), constant_values=N).reshape(EP // CH, CH)


def kernel(features, edge_index, CL_graph, W1, W2, att_src1, att_dst1):
    featp = jnp.pad(features, ((0, NP - N), (0, 0)))
    src2d = _pad_edges(edge_index[0].astype(_i32))
    dst2d = _pad_edges(edge_index[1].astype(_i32))

    a_src, a_dst = _tc1(featp, W1,
                        att_src1.reshape(1, HID), att_dst1.reshape(1, HID))

    ex2d, dparts = _b1_kernel()(src2d, dst2d,
                                a_src.reshape(NP), a_dst.reshape(NP))

    featsplit = jnp.concatenate([featp[:, :64], featp[:, 64:]], axis=0)
    acc1 = _agg_kernel(64, EP // CH // NS, True)(
        src2d, dst2d, ex2d, dparts, featsplit)
    af = jnp.concatenate([acc1[:NP], acc1[NP:]], axis=1)
    h2p = _tc3(af, W1, W2)

    acc3 = _agg_kernel(OUT_DIM, EP // CH // (NC * NS), False)(
        src2d, dst2d, ex2d, dparts, h2p)
    ah_parts = acc3.reshape(NC, NP, OUT_DIM)[:, :N]

    h4, pi, disp, mean = _tc4(ah_parts, W2, W1)

    gat = jnp.concatenate([_pad_edges(edge_index[1].astype(_i32)),
                           _pad_edges(CL_graph[1].astype(_i32))], axis=0)
    sct = jnp.concatenate([_pad_edges(edge_index[0].astype(_i32)),
                           _pad_edges(CL_graph[0].astype(_i32))], axis=0)
    csl = _csl_kernel()(gat, sct, h2p)

    return (h2p[:N], csl[:N], csl[NP:NP + N], mean, disp, pi, h4)


# speedup vs baseline: 1.1157x; 1.1157x over previous
import functools

import jax
import jax.numpy as jnp
from jax import lax
from jax.experimental import pallas as pl
from jax.experimental.pallas import tpu as pltpu
from jax.experimental.pallas import tpu_sc as plsc

N = 10000
E = 160000
IN_DIM = 128
HID = 256
OUT_DIM = 32

NP = 10240
EP = 163840
NC = 2
NS = 16
L = 16
CH = 128
ROWS_B1 = EP // (NC * NS) // CH
ROWS_AGG = EP // NS // CH
NPT = NP // NS

_f32 = jnp.float32
_i32 = jnp.int32


def _elu(x):
    return jnp.where(x > 0, x, jnp.exp(jnp.minimum(x, 0.0)) - 1.0)


def _tc1_body(feat_ref, w1_ref, att_ref, ab_ref):
    w1a = jnp.dot(w1_ref[...], att_ref[...], preferred_element_type=_f32)
    ab_ref[...] = jnp.dot(feat_ref[...], w1a, preferred_element_type=_f32)


def _tc1(featp, w1, att2):
    R = 1024
    grid = (NP // R,)
    return pl.pallas_call(
        _tc1_body,
        grid=grid,
        in_specs=[
            pl.BlockSpec((R, IN_DIM), lambda i: (i, 0)),
            pl.BlockSpec((IN_DIM, HID), lambda i: (0, 0)),
            pl.BlockSpec((HID, NC), lambda i: (0, 0)),
        ],
        out_specs=pl.BlockSpec((R, NC), lambda i: (i, 0)),
        out_shape=jax.ShapeDtypeStruct((NP, NC), _f32),
    )(featp, w1, att2)


def _tc2_body(dp_ref, out_ref):
    out_ref[...] = 1.0 / (dp_ref[0] + dp_ref[1] + 1e-16)


def _tc2(denom_parts3):
    return pl.pallas_call(
        _tc2_body,
        out_shape=jax.ShapeDtypeStruct((NP // 128, 128), _f32),
    )(denom_parts3)


def _tc3_body(af_ref, w1_ref, w2_ref, h2_ref):
    h1 = _elu(jnp.dot(af_ref[...], w1_ref[...], preferred_element_type=_f32))
    h2_ref[...] = jnp.dot(h1, w2_ref[...], preferred_element_type=_f32)


def _tc3(af, w1, w2):
    R = 512
    grid = (NP // R,)
    return pl.pallas_call(
        _tc3_body,
        grid=grid,
        in_specs=[
            pl.BlockSpec((R, IN_DIM), lambda i: (i, 0)),
            pl.BlockSpec((IN_DIM, HID), lambda i: (0, 0)),
            pl.BlockSpec((HID, OUT_DIM), lambda i: (0, 0)),
        ],
        out_specs=pl.BlockSpec((R, OUT_DIM), lambda i: (i, 0)),
        out_shape=jax.ShapeDtypeStruct((NP, OUT_DIM), _f32),
    )(af, w1, w2)


def _tc4_body(ah_ref, w2_ref, w1_ref, h4_ref, pi_ref, disp_ref, mean_ref):
    ah = ah_ref[0] + ah_ref[1]
    t = lax.dot_general(ah, w2_ref[...], (((1,), (1,)), ((), ())),
                        preferred_element_type=_f32)
    h3 = _elu(t)
    h4 = lax.dot_general(h3, w1_ref[...], (((1,), (1,)), ((), ())),
                         preferred_element_type=_f32)
    h4_ref[...] = h4
    pi_ref[...] = 1.0 / (1.0 + jnp.exp(-h4))
    sp = jnp.log(1.0 + jnp.exp(-jnp.abs(h4))) + jnp.maximum(h4, 0.0)
    disp_ref[...] = jnp.clip(sp, 0.0001, 10000.0)
    mean_ref[...] = jnp.clip(jnp.exp(h4), 1e-05, 1000000.0)


def _tc4(ah_parts, w2, w1):
    R = 1000
    grid = (N // R,)
    o = jax.ShapeDtypeStruct((N, IN_DIM), _f32)
    return pl.pallas_call(
        _tc4_body,
        grid=grid,
        in_specs=[
            pl.BlockSpec((NC, R, OUT_DIM), lambda i: (0, i, 0)),
            pl.BlockSpec((HID, OUT_DIM), lambda i: (0, 0)),
            pl.BlockSpec((IN_DIM, HID), lambda i: (0, 0)),
        ],
        out_specs=[pl.BlockSpec((R, IN_DIM), lambda i: (i, 0))] * 4,
        out_shape=[o, o, o, o],
    )(ah_parts, w2, w1)


@functools.lru_cache(maxsize=None)
def _mesh():
    return plsc.VectorSubcoreMesh(core_axis_name="c", subcore_axis_name="s",
                                  num_cores=NC, num_subcores=NS)


def _zero_vmem(ref, nrows):
    z = jnp.zeros((L,), _f32)

    def body(j, _):
        for q in range(CH // L):
            ref[j, pl.ds(q * L, L)] = z
        return 0

    lax.fori_loop(0, nrows, body, 0)


def _b1_body(src_hbm, dst_hbm, asrc_hbm, adst_hbm, ex_hbm, dpart_hbm,
             asv, adv, srcv, dstv, exv, zbuf, denom_sh):
    c = lax.axis_index("c")
    s = lax.axis_index("s")
    wid = s * NC + c

    pltpu.sync_copy(asrc_hbm, asv)
    pltpu.sync_copy(adst_hbm, adv)
    pltpu.sync_copy(src_hbm.at[pl.ds(wid * ROWS_B1, ROWS_B1)], srcv)
    pltpu.sync_copy(dst_hbm.at[pl.ds(wid * ROWS_B1, ROWS_B1)], dstv)

    z = jnp.zeros((L,), _f32)
    def zb(j, _):
        zbuf[pl.ds(j * L, L)] = z
        return 0
    lax.fori_loop(0, NPT // L, zb, 0)
    pltpu.sync_copy(zbuf, denom_sh.at[pl.ds(s * NPT, NPT)])
    plsc.subcore_barrier()

    def chunk(j, _):
        for q in range(CH // L):
            sv = srcv[j, pl.ds(q * L, L)]
            dv = dstv[j, pl.ds(q * L, L)]
            a1 = plsc.load_gather(asv, [sv])
            a2 = plsc.load_gather(adv, [dv])
            x = a1 + a2
            sig = 1.0 / (1.0 + jnp.exp(-x))
            exv[j, pl.ds(q * L, L)] = jnp.exp(sig)
        return 0

    lax.fori_loop(0, ROWS_B1, chunk, 0)

    def scat(j, _):
        pltpu.sync_copy(exv.at[j], denom_sh.at[dstv.at[j]], add=True)
        return 0

    lax.fori_loop(0, ROWS_B1, scat, 0)
    pltpu.sync_copy(exv, ex_hbm.at[pl.ds(wid * ROWS_B1, ROWS_B1)])
    plsc.subcore_barrier()

    pltpu.sync_copy(denom_sh.at[pl.ds(s * NPT, NPT)],
                    dpart_hbm.at[pl.ds(c * NP + s * NPT, NPT)])


@functools.lru_cache(maxsize=None)
def _b1_kernel():
    return pl.kernel(
        _b1_body,
        out_type=[
            jax.ShapeDtypeStruct((EP // CH, CH), _f32),
            jax.ShapeDtypeStruct((NC * NP,), _f32),
        ],
        mesh=_mesh(),
        compiler_params=pltpu.CompilerParams(use_tc_tiling_on_sc=False, needs_layout_passes=False),
        scratch_types=[
            pltpu.VMEM((NP,), _f32),
            pltpu.VMEM((NP,), _f32),
            pltpu.VMEM((ROWS_B1, CH), _i32),
            pltpu.VMEM((ROWS_B1, CH), _i32),
            pltpu.VMEM((ROWS_B1, CH), _f32),
            pltpu.VMEM((NPT,), _f32),
            pltpu.VMEM_SHARED((NP,), _f32),
        ],
    )


G_AGG = 10


def _make_agg_body(d, chunks_per_tile, dim_split):
    n_grp = chunks_per_tile // G_AGG
    qd = d // L

    def body(src_hbm, dst_hbm, ex_hbm, dpart_hbm, table_hbm, out_hbm,
             srcv, dstv, exv, invdv, wv, rows0, rows1,
             gsem0, gsem1, ssem0, ssem1, accum_sh):
        c = lax.axis_index("c")
        s = lax.axis_index("s")

        pltpu.sync_copy(dpart_hbm, invdv)

        if dim_split:
            off = (c * NP).astype(_i32)
            tile_chunk0 = s * chunks_per_tile
        else:
            off = None
            tile_chunk0 = (c * NS + s) * chunks_per_tile

        z = jnp.zeros((L,), _f32)

        def zrow(e, _):
            for q in range(qd):
                rows0[e, pl.ds(q * L, L)] = z
            return 0
        lax.fori_loop(0, CH, zrow, 0)
        for r0 in range(0, NPT, CH):
            pltpu.sync_copy(rows0, accum_sh.at[pl.ds(s * NPT + r0, CH)])
        plsc.subcore_barrier()

        bufs = ((rows0, gsem0, ssem0), (rows1, gsem1, ssem1))

        def group(g, _):
            base = tile_chunk0 + g * G_AGG
            pltpu.sync_copy(src_hbm.at[pl.ds(base, G_AGG)], srcv)
            pltpu.sync_copy(dst_hbm.at[pl.ds(base, G_AGG)], dstv)
            pltpu.sync_copy(ex_hbm.at[pl.ds(base, G_AGG)], exv)

            for j in range(G_AGG):
                for q in range(CH // L):
                    dv = dstv[j, pl.ds(q * L, L)]
                    iv = plsc.load_gather(invdv, [dv])
                    wv[j, pl.ds(q * L, L)] = exv[j, pl.ds(q * L, L)] * iv
                    if off is not None:
                        srcv[j, pl.ds(q * L, L)] = (
                            srcv[j, pl.ds(q * L, L)] + off)

            gd = [None] * G_AGG
            sd = [None] * G_AGG
            gd[0] = pltpu.async_copy(table_hbm.at[srcv.at[0]], rows0, gsem0)
            for j in range(G_AGG):
                rb, _, ssb = bufs[j & 1]
                ro, gso, _ = bufs[(j + 1) & 1]
                if j + 1 < G_AGG:
                    if j >= 1:
                        sd[j - 1].wait()
                    gd[j + 1] = pltpu.async_copy(
                        table_hbm.at[srcv.at[j + 1]], ro, gso)
                gd[j].wait()

                @pl.loop(0, CH, unroll=4)
                def _scale(e):
                    wbc = plsc.load_gather(
                        wv, [jnp.full((L,), j, _i32), jnp.full((L,), e, _i32)])
                    for q in range(qd):
                        rb[e, pl.ds(q * L, L)] = rb[e, pl.ds(q * L, L)] * wbc

                sd[j] = pltpu.async_copy(rb, accum_sh.at[dstv.at[j]], ssb,
                                         add=True)
            sd[G_AGG - 2].wait()
            sd[G_AGG - 1].wait()
            return 0

        lax.fori_loop(0, n_grp, group, 0)
        plsc.subcore_barrier()
        pltpu.sync_copy(accum_sh.at[pl.ds(s * NPT, NPT)],
                        out_hbm.at[pl.ds(c * NP + s * NPT, NPT)])

    return body


@functools.lru_cache(maxsize=None)
def _agg_kernel(d, chunks_per_tile, dim_split):
    return pl.kernel(
        _make_agg_body(d, chunks_per_tile, dim_split),
        out_type=jax.ShapeDtypeStruct((NC * NP, d), _f32),
        mesh=_mesh(),
        compiler_params=pltpu.CompilerParams(use_tc_tiling_on_sc=False, needs_layout_passes=False),
        scratch_types=[
            pltpu.VMEM((G_AGG, CH), _i32),
            pltpu.VMEM((G_AGG, CH), _i32),
            pltpu.VMEM((G_AGG, CH), _f32),
            pltpu.VMEM((NP,), _f32),
            pltpu.VMEM((G_AGG, CH), _f32),
            pltpu.VMEM((CH, d), _f32),
            pltpu.VMEM((CH, d), _f32),
            pltpu.SemaphoreType.DMA,
            pltpu.SemaphoreType.DMA,
            pltpu.SemaphoreType.DMA,
            pltpu.SemaphoreType.DMA,
            pltpu.VMEM_SHARED((NP, d), _f32),
        ],
    )


def _csl_body(gat_hbm, sct_hbm, h2_hbm, out_hbm,
              gv, sv, onesv, rows0, rows1, cntv, mv,
              gsem0, gsem1, ssem0, ssem1, csem, accum_sh, cnt_sh):
    c = lax.axis_index("c")
    s = lax.axis_index("s")
    base = c * (EP // CH) + s * ROWS_CSL

    pltpu.sync_copy(gat_hbm.at[pl.ds(base, ROWS_CSL)], gv)
    pltpu.sync_copy(sct_hbm.at[pl.ds(base, ROWS_CSL)], sv)

    one = jnp.full((L,), 1.0, _f32)
    z = jnp.zeros((L,), _f32)
    for q in range(CH // L):
        onesv[pl.ds(q * L, L)] = one

    def zrow(e, _):
        for q in range(OUT_DIM // L):
            rows0[e, pl.ds(q * L, L)] = z
        return 0
    lax.fori_loop(0, CH, zrow, 0)

    for r0 in range(0, NPT, CH):
        pltpu.sync_copy(rows0, accum_sh.at[pl.ds(s * NPT + r0, CH)])
    def zb(k, _):
        cntv[pl.ds(k * L, L)] = z
        return 0
    lax.fori_loop(0, NPT // L, zb, 0)
    pltpu.sync_copy(cntv, cnt_sh.at[pl.ds(s * NPT, NPT)])
    plsc.subcore_barrier()

    bufs = ((rows0, gsem0, ssem0), (rows1, gsem1, ssem1))
    gd = [None] * ROWS_CSL
    sd = [None] * ROWS_CSL
    cd = [None] * ROWS_CSL
    gd[0] = pltpu.async_copy(h2_hbm.at[gv.at[0]], rows0, gsem0)
    for j in range(ROWS_CSL):
        rb, _, ssb = bufs[j & 1]
        ro, gso, _ = bufs[(j + 1) & 1]
        if j + 1 < ROWS_CSL:
            if j >= 1:
                sd[j - 1].wait()
            gd[j + 1] = pltpu.async_copy(h2_hbm.at[gv.at[j + 1]], ro, gso)
        gd[j].wait()
        sd[j] = pltpu.async_copy(rb, accum_sh.at[sv.at[j]], ssb, add=True)
        cd[j] = pltpu.async_copy(onesv, cnt_sh.at[sv.at[j]], csem, add=True)
        if j >= 1:
            cd[j - 1].wait()
    sd[ROWS_CSL - 2].wait()
    sd[ROWS_CSL - 1].wait()
    cd[ROWS_CSL - 1].wait()
    plsc.subcore_barrier()

    pltpu.sync_copy(cnt_sh.at[pl.ds(s * NPT, NPT)], cntv)
    pltpu.sync_copy(accum_sh.at[pl.ds(s * NPT, NPT)], mv)

    def div(r, _):
        cb = plsc.load_gather(cntv, [jnp.full((L,), r, _i32)])
        rec = 1.0 / jnp.maximum(cb, 1.0)
        for q in range(OUT_DIM // L):
            mv[r, pl.ds(q * L, L)] = mv[r, pl.ds(q * L, L)] * rec
        return 0

    lax.fori_loop(0, NPT, div, 0)
    pltpu.sync_copy(mv, out_hbm.at[pl.ds(c * NP + s * NPT, NPT)])


ROWS_CSL = EP // NS // CH


@functools.lru_cache(maxsize=None)
def _csl_kernel():
    return pl.kernel(
        _csl_body,
        out_type=jax.ShapeDtypeStruct((NC * NP, OUT_DIM), _f32),
        mesh=_mesh(),
        compiler_params=pltpu.CompilerParams(use_tc_tiling_on_sc=False, needs_layout_passes=False),
        scratch_types=[
            pltpu.VMEM((ROWS_CSL, CH), _i32),
            pltpu.VMEM((ROWS_CSL, CH), _i32),
            pltpu.VMEM((CH,), _f32),
            pltpu.VMEM((CH, OUT_DIM), _f32),
            pltpu.VMEM((CH, OUT_DIM), _f32),
            pltpu.VMEM((NPT,), _f32),
            pltpu.VMEM((NPT, OUT_DIM), _f32),
            pltpu.SemaphoreType.DMA,
            pltpu.SemaphoreType.DMA,
            pltpu.SemaphoreType.DMA,
            pltpu.SemaphoreType.DMA,
            pltpu.SemaphoreType.DMA,
            pltpu.VMEM_SHARED((NP, OUT_DIM), _f32),
            pltpu.VMEM_SHARED((NP,), _f32),
        ],
    )


def _pad_edges(idx):
    return jnp.pad(idx, (0, EP - E), constant_values=N).reshape(EP // CH, CH)


def kernel(features, edge_index, CL_graph, W1, W2, att_src1, att_dst1):
    featp = jnp.pad(features, ((0, NP - N), (0, 0)))
    src2d = _pad_edges(edge_index[0].astype(_i32))
    dst2d = _pad_edges(edge_index[1].astype(_i32))

    ab = _tc1(featp, W1, jnp.stack([att_src1, att_dst1], axis=1))

    ex2d, dparts = _b1_kernel()(src2d, dst2d, ab[:, 0], ab[:, 1])
    invd = _tc2(dparts.reshape(NC, NP // 128, 128)).reshape(NP)

    featsplit = jnp.concatenate([featp[:, :64], featp[:, 64:]], axis=0)
    acc1 = _agg_kernel(64, EP // CH // NS, True)(
        src2d, dst2d, ex2d, invd, featsplit)
    af = jnp.concatenate([acc1[:NP], acc1[NP:]], axis=1)
    h2p = _tc3(af, W1, W2)

    acc3 = _agg_kernel(OUT_DIM, EP // CH // (NC * NS), False)(
        src2d, dst2d, ex2d, invd, h2p)
    ah_parts = acc3.reshape(NC, NP, OUT_DIM)[:, :N]

    h4, pi, disp, mean = _tc4(ah_parts, W2, W1)

    gat = jnp.concatenate([_pad_edges(edge_index[1].astype(_i32)),
                           _pad_edges(CL_graph[1].astype(_i32))], axis=0)
    sct = jnp.concatenate([_pad_edges(edge_index[0].astype(_i32)),
                           _pad_edges(CL_graph[0].astype(_i32))], axis=0)
    csl = _csl_kernel()(gat, sct, h2p)

    return (h2p[:N], csl[:N], csl[NP:NP + N], mean, disp, pi, h4)


# speedup vs baseline: 1.1418x; 1.0234x over previous
import functools

import jax
import jax.numpy as jnp
from jax import lax
from jax.experimental import pallas as pl
from jax.experimental.pallas import tpu as pltpu
from jax.experimental.pallas import tpu_sc as plsc

N = 10000
E = 160000
IN_DIM = 128
HID = 256
OUT_DIM = 32

NP = 10240
EP = 163840
NC = 2
NS = 16
L = 16
CH = 128
ROWS_B1 = EP // (NC * NS) // CH
ROWS_AGG = EP // NS // CH
NPT = NP // NS

_f32 = jnp.float32
_i32 = jnp.int32


def _elu(x):
    return jnp.where(x > 0, x, jnp.exp(jnp.minimum(x, 0.0)) - 1.0)


def _tc1_body(feat_ref, w1_ref, att_ref, ab_ref):
    w1a = jnp.dot(w1_ref[...], att_ref[...], preferred_element_type=_f32)
    ab_ref[...] = jnp.dot(feat_ref[...], w1a, preferred_element_type=_f32)


def _tc1(featp, w1, att2):
    R = 1024
    grid = (NP // R,)
    return pl.pallas_call(
        _tc1_body,
        grid=grid,
        in_specs=[
            pl.BlockSpec((R, IN_DIM), lambda i: (i, 0)),
            pl.BlockSpec((IN_DIM, HID), lambda i: (0, 0)),
            pl.BlockSpec((HID, NC), lambda i: (0, 0)),
        ],
        out_specs=pl.BlockSpec((R, NC), lambda i: (i, 0)),
        out_shape=jax.ShapeDtypeStruct((NP, NC), _f32),
    )(featp, w1, att2)


def _tc2_body(dp_ref, out_ref):
    out_ref[...] = 1.0 / (dp_ref[0] + dp_ref[1] + 1e-16)


def _tc2(denom_parts3):
    return pl.pallas_call(
        _tc2_body,
        out_shape=jax.ShapeDtypeStruct((NP // 128, 128), _f32),
    )(denom_parts3)


def _tc3_body(af_ref, w1_ref, w2_ref, h2_ref):
    h1 = _elu(jnp.dot(af_ref[...], w1_ref[...], preferred_element_type=_f32))
    h2_ref[...] = jnp.dot(h1, w2_ref[...], preferred_element_type=_f32)


def _tc3(af, w1, w2):
    R = 512
    grid = (NP // R,)
    return pl.pallas_call(
        _tc3_body,
        grid=grid,
        in_specs=[
            pl.BlockSpec((R, IN_DIM), lambda i: (i, 0)),
            pl.BlockSpec((IN_DIM, HID), lambda i: (0, 0)),
            pl.BlockSpec((HID, OUT_DIM), lambda i: (0, 0)),
        ],
        out_specs=pl.BlockSpec((R, OUT_DIM), lambda i: (i, 0)),
        out_shape=jax.ShapeDtypeStruct((NP, OUT_DIM), _f32),
    )(af, w1, w2)


def _tc4_body(ah_ref, w2_ref, w1_ref, h4_ref, pi_ref, disp_ref, mean_ref):
    ah = ah_ref[0] + ah_ref[1]
    t = lax.dot_general(ah, w2_ref[...], (((1,), (1,)), ((), ())),
                        preferred_element_type=_f32)
    h3 = _elu(t)
    h4 = lax.dot_general(h3, w1_ref[...], (((1,), (1,)), ((), ())),
                         preferred_element_type=_f32)
    h4_ref[...] = h4
    pi_ref[...] = 1.0 / (1.0 + jnp.exp(-h4))
    sp = jnp.log(1.0 + jnp.exp(-jnp.abs(h4))) + jnp.maximum(h4, 0.0)
    disp_ref[...] = jnp.clip(sp, 0.0001, 10000.0)
    mean_ref[...] = jnp.clip(jnp.exp(h4), 1e-05, 1000000.0)


def _tc4(ah_parts, w2, w1):
    R = 1000
    grid = (N // R,)
    o = jax.ShapeDtypeStruct((N, IN_DIM), _f32)
    return pl.pallas_call(
        _tc4_body,
        grid=grid,
        in_specs=[
            pl.BlockSpec((NC, R, OUT_DIM), lambda i: (0, i, 0)),
            pl.BlockSpec((HID, OUT_DIM), lambda i: (0, 0)),
            pl.BlockSpec((IN_DIM, HID), lambda i: (0, 0)),
        ],
        out_specs=[pl.BlockSpec((R, IN_DIM), lambda i: (i, 0))] * 4,
        out_shape=[o, o, o, o],
    )(ah_parts, w2, w1)


@functools.lru_cache(maxsize=None)
def _mesh():
    return plsc.VectorSubcoreMesh(core_axis_name="c", subcore_axis_name="s",
                                  num_cores=NC, num_subcores=NS)


def _zero_vmem(ref, nrows):
    z = jnp.zeros((L,), _f32)

    def body(j, _):
        for q in range(CH // L):
            ref[j, pl.ds(q * L, L)] = z
        return 0

    lax.fori_loop(0, nrows, body, 0)


def _b1_body(src_hbm, dst_hbm, asrc_hbm, adst_hbm, ex_hbm, dpart_hbm,
             asv, adv, srcv, dstv, exv, zbuf, denom_sh):
    c = lax.axis_index("c")
    s = lax.axis_index("s")
    wid = s * NC + c

    pltpu.sync_copy(asrc_hbm, asv)
    pltpu.sync_copy(adst_hbm, adv)
    pltpu.sync_copy(src_hbm.at[pl.ds(wid * ROWS_B1, ROWS_B1)], srcv)
    pltpu.sync_copy(dst_hbm.at[pl.ds(wid * ROWS_B1, ROWS_B1)], dstv)

    z = jnp.zeros((L,), _f32)
    def zb(j, _):
        zbuf[pl.ds(j * L, L)] = z
        return 0
    lax.fori_loop(0, NPT // L, zb, 0)
    pltpu.sync_copy(zbuf, denom_sh.at[pl.ds(s * NPT, NPT)])
    plsc.subcore_barrier()

    def chunk(j, _):
        for q in range(CH // L):
            sv = srcv[j, pl.ds(q * L, L)]
            dv = dstv[j, pl.ds(q * L, L)]
            a1 = plsc.load_gather(asv, [sv])
            a2 = plsc.load_gather(adv, [dv])
            x = a1 + a2
            sig = 1.0 / (1.0 + jnp.exp(-x))
            exv[j, pl.ds(q * L, L)] = jnp.exp(sig)
        return 0

    lax.fori_loop(0, ROWS_B1, chunk, 0)

    def scat(j, _):
        pltpu.sync_copy(exv.at[j], denom_sh.at[dstv.at[j]], add=True)
        return 0

    lax.fori_loop(0, ROWS_B1, scat, 0)
    pltpu.sync_copy(exv, ex_hbm.at[pl.ds(wid * ROWS_B1, ROWS_B1)])
    plsc.subcore_barrier()

    pltpu.sync_copy(denom_sh.at[pl.ds(s * NPT, NPT)],
                    dpart_hbm.at[pl.ds(c * NP + s * NPT, NPT)])


@functools.lru_cache(maxsize=None)
def _b1_kernel():
    return pl.kernel(
        _b1_body,
        out_type=[
            jax.ShapeDtypeStruct((EP // CH, CH), _f32),
            jax.ShapeDtypeStruct((NC * NP,), _f32),
        ],
        mesh=_mesh(),
        compiler_params=pltpu.CompilerParams(use_tc_tiling_on_sc=False, needs_layout_passes=False),
        scratch_types=[
            pltpu.VMEM((NP,), _f32),
            pltpu.VMEM((NP,), _f32),
            pltpu.VMEM((ROWS_B1, CH), _i32),
            pltpu.VMEM((ROWS_B1, CH), _i32),
            pltpu.VMEM((ROWS_B1, CH), _f32),
            pltpu.VMEM((NPT,), _f32),
            pltpu.VMEM_SHARED((NP,), _f32),
        ],
    )


G_AGG = 20


def _make_agg_body(d, chunks_per_tile, dim_split):
    n_grp = chunks_per_tile // G_AGG
    qd = d // L

    def body(src_hbm, dst_hbm, ex_hbm, dpart_hbm, table_hbm, out_hbm,
             srcv, dstv, exv, invdv, wv, rows0, rows1,
             gsem0, gsem1, ssem0, ssem1, accum_sh):
        c = lax.axis_index("c")
        s = lax.axis_index("s")

        pltpu.sync_copy(dpart_hbm, invdv)

        if dim_split:
            off = (c * NP).astype(_i32)
            tile_chunk0 = s * chunks_per_tile
        else:
            off = None
            tile_chunk0 = (c * NS + s) * chunks_per_tile

        z = jnp.zeros((L,), _f32)

        def zrow(e, _):
            for q in range(qd):
                rows0[e, pl.ds(q * L, L)] = z
            return 0
        lax.fori_loop(0, CH, zrow, 0)
        for r0 in range(0, NPT, CH):
            pltpu.sync_copy(rows0, accum_sh.at[pl.ds(s * NPT + r0, CH)])
        plsc.subcore_barrier()

        bufs = ((rows0, gsem0, ssem0), (rows1, gsem1, ssem1))

        def group(g, _):
            base = tile_chunk0 + g * G_AGG
            pltpu.sync_copy(src_hbm.at[pl.ds(base, G_AGG)], srcv)
            pltpu.sync_copy(dst_hbm.at[pl.ds(base, G_AGG)], dstv)
            pltpu.sync_copy(ex_hbm.at[pl.ds(base, G_AGG)], exv)

            for j in range(G_AGG):
                for q in range(CH // L):
                    dv = dstv[j, pl.ds(q * L, L)]
                    iv = plsc.load_gather(invdv, [dv])
                    wv[j, pl.ds(q * L, L)] = exv[j, pl.ds(q * L, L)] * iv
                    if off is not None:
                        srcv[j, pl.ds(q * L, L)] = (
                            srcv[j, pl.ds(q * L, L)] + off)

            gd = [None] * G_AGG
            sd = [None] * G_AGG
            gd[0] = pltpu.async_copy(table_hbm.at[srcv.at[0]], rows0, gsem0)
            for j in range(G_AGG):
                rb, _, ssb = bufs[j & 1]
                ro, gso, _ = bufs[(j + 1) & 1]
                if j + 1 < G_AGG:
                    if j >= 1:
                        sd[j - 1].wait()
                    gd[j + 1] = pltpu.async_copy(
                        table_hbm.at[srcv.at[j + 1]], ro, gso)
                gd[j].wait()

                @pl.loop(0, CH, unroll=4)
                def _scale(e):
                    wbc = plsc.load_gather(
                        wv, [jnp.full((L,), j, _i32), jnp.full((L,), e, _i32)])
                    for q in range(qd):
                        rb[e, pl.ds(q * L, L)] = rb[e, pl.ds(q * L, L)] * wbc

                sd[j] = pltpu.async_copy(rb, accum_sh.at[dstv.at[j]], ssb,
                                         add=True)
            sd[G_AGG - 2].wait()
            sd[G_AGG - 1].wait()
            return 0

        lax.fori_loop(0, n_grp, group, 0)
        plsc.subcore_barrier()
        pltpu.sync_copy(accum_sh.at[pl.ds(s * NPT, NPT)],
                        out_hbm.at[pl.ds(c * NP + s * NPT, NPT)])

    return body


@functools.lru_cache(maxsize=None)
def _agg_kernel(d, chunks_per_tile, dim_split):
    return pl.kernel(
        _make_agg_body(d, chunks_per_tile, dim_split),
        out_type=jax.ShapeDtypeStruct((NC * NP, d), _f32),
        mesh=_mesh(),
        compiler_params=pltpu.CompilerParams(use_tc_tiling_on_sc=False, needs_layout_passes=False),
        scratch_types=[
            pltpu.VMEM((G_AGG, CH), _i32),
            pltpu.VMEM((G_AGG, CH), _i32),
            pltpu.VMEM((G_AGG, CH), _f32),
            pltpu.VMEM((NP,), _f32),
            pltpu.VMEM((G_AGG, CH), _f32),
            pltpu.VMEM((CH, d), _f32),
            pltpu.VMEM((CH, d), _f32),
            pltpu.SemaphoreType.DMA,
            pltpu.SemaphoreType.DMA,
            pltpu.SemaphoreType.DMA,
            pltpu.SemaphoreType.DMA,
            pltpu.VMEM_SHARED((NP, d), _f32),
        ],
    )


def _csl_body(gat_hbm, sct_hbm, h2_hbm, out_hbm,
              gv, sv, onesv, rows0, rows1, cntv, mv,
              gsem0, gsem1, ssem0, ssem1, csem, accum_sh, cnt_sh):
    c = lax.axis_index("c")
    s = lax.axis_index("s")
    base = c * (EP // CH) + s * ROWS_CSL

    pltpu.sync_copy(gat_hbm.at[pl.ds(base, ROWS_CSL)], gv)
    pltpu.sync_copy(sct_hbm.at[pl.ds(base, ROWS_CSL)], sv)

    one = jnp.full((L,), 1.0, _f32)
    z = jnp.zeros((L,), _f32)
    for q in range(CH // L):
        onesv[pl.ds(q * L, L)] = one

    def zrow(e, _):
        for q in range(OUT_DIM // L):
            rows0[e, pl.ds(q * L, L)] = z
        return 0
    lax.fori_loop(0, CH, zrow, 0)

    for r0 in range(0, NPT, CH):
        pltpu.sync_copy(rows0, accum_sh.at[pl.ds(s * NPT + r0, CH)])
    def zb(k, _):
        cntv[pl.ds(k * L, L)] = z
        return 0
    lax.fori_loop(0, NPT // L, zb, 0)
    pltpu.sync_copy(cntv, cnt_sh.at[pl.ds(s * NPT, NPT)])
    plsc.subcore_barrier()

    bufs = ((rows0, gsem0, ssem0), (rows1, gsem1, ssem1))
    gd = [None] * ROWS_CSL
    sd = [None] * ROWS_CSL
    cd = [None] * ROWS_CSL
    gd[0] = pltpu.async_copy(h2_hbm.at[gv.at[0]], rows0, gsem0)
    for j in range(ROWS_CSL):
        rb, _, ssb = bufs[j & 1]
        ro, gso, _ = bufs[(j + 1) & 1]
        if j + 1 < ROWS_CSL:
            if j >= 1:
                sd[j - 1].wait()
            gd[j + 1] = pltpu.async_copy(h2_hbm.at[gv.at[j + 1]], ro, gso)
        gd[j].wait()
        sd[j] = pltpu.async_copy(rb, accum_sh.at[sv.at[j]], ssb, add=True)
        cd[j] = pltpu.async_copy(onesv, cnt_sh.at[sv.at[j]], csem, add=True)
        if j >= 1:
            cd[j - 1].wait()
    sd[ROWS_CSL - 2].wait()
    sd[ROWS_CSL - 1].wait()
    cd[ROWS_CSL - 1].wait()
    plsc.subcore_barrier()

    pltpu.sync_copy(cnt_sh.at[pl.ds(s * NPT, NPT)], cntv)
    pltpu.sync_copy(accum_sh.at[pl.ds(s * NPT, NPT)], mv)

    def div(r, _):
        cb = plsc.load_gather(cntv, [jnp.full((L,), r, _i32)])
        rec = 1.0 / jnp.maximum(cb, 1.0)
        for q in range(OUT_DIM // L):
            mv[r, pl.ds(q * L, L)] = mv[r, pl.ds(q * L, L)] * rec
        return 0

    lax.fori_loop(0, NPT, div, 0)
    pltpu.sync_copy(mv, out_hbm.at[pl.ds(c * NP + s * NPT, NPT)])


ROWS_CSL = EP // NS // CH


@functools.lru_cache(maxsize=None)
def _csl_kernel():
    return pl.kernel(
        _csl_body,
        out_type=jax.ShapeDtypeStruct((NC * NP, OUT_DIM), _f32),
        mesh=_mesh(),
        compiler_params=pltpu.CompilerParams(use_tc_tiling_on_sc=False, needs_layout_passes=False),
        scratch_types=[
            pltpu.VMEM((ROWS_CSL, CH), _i32),
            pltpu.VMEM((ROWS_CSL, CH), _i32),
            pltpu.VMEM((CH,), _f32),
            pltpu.VMEM((CH, OUT_DIM), _f32),
            pltpu.VMEM((CH, OUT_DIM), _f32),
            pltpu.VMEM((NPT,), _f32),
            pltpu.VMEM((NPT, OUT_DIM), _f32),
            pltpu.SemaphoreType.DMA,
            pltpu.SemaphoreType.DMA,
            pltpu.SemaphoreType.DMA,
            pltpu.SemaphoreType.DMA,
            pltpu.SemaphoreType.DMA,
            pltpu.VMEM_SHARED((NP, OUT_DIM), _f32),
            pltpu.VMEM_SHARED((NP,), _f32),
        ],
    )


def _pad_edges(idx):
    return jnp.pad(idx, (0, EP - E), constant_values=N).reshape(EP // CH, CH)


def kernel(features, edge_index, CL_graph, W1, W2, att_src1, att_dst1):
    featp = jnp.pad(features, ((0, NP - N), (0, 0)))
    src2d = _pad_edges(edge_index[0].astype(_i32))
    dst2d = _pad_edges(edge_index[1].astype(_i32))

    ab = _tc1(featp, W1, jnp.stack([att_src1, att_dst1], axis=1))

    ex2d, dparts = _b1_kernel()(src2d, dst2d, ab[:, 0], ab[:, 1])
    invd = _tc2(dparts.reshape(NC, NP // 128, 128)).reshape(NP)

    featsplit = jnp.concatenate([featp[:, :64], featp[:, 64:]], axis=0)
    acc1 = _agg_kernel(64, EP // CH // NS, True)(
        src2d, dst2d, ex2d, invd, featsplit)
    af = jnp.concatenate([acc1[:NP], acc1[NP:]], axis=1)
    h2p = _tc3(af, W1, W2)

    acc3 = _agg_kernel(OUT_DIM, EP // CH // (NC * NS), False)(
        src2d, dst2d, ex2d, invd, h2p)
    ah_parts = acc3.reshape(NC, NP, OUT_DIM)[:, :N]

    h4, pi, disp, mean = _tc4(ah_parts, W2, W1)

    gat = jnp.concatenate([_pad_edges(edge_index[1].astype(_i32)),
                           _pad_edges(CL_graph[1].astype(_i32))], axis=0)
    sct = jnp.concatenate([_pad_edges(edge_index[0].astype(_i32)),
                           _pad_edges(CL_graph[0].astype(_i32))], axis=0)
    csl = _csl_kernel()(gat, sct, h2p)

    return (h2p[:N], csl[:N], csl[NP:NP + N], mean, disp, pi, h4)


# speedup vs baseline: 1.1492x; 1.0064x over previous
import functools

import jax
import jax.numpy as jnp
from jax import lax
from jax.experimental import pallas as pl
from jax.experimental.pallas import tpu as pltpu
from jax.experimental.pallas import tpu_sc as plsc

N = 10000
E = 160000
IN_DIM = 128
HID = 256
OUT_DIM = 32

NP = 10240
EP = 163840
NC = 2
NS = 16
L = 16
CH = 128
ROWS_B1 = EP // (NC * NS) // CH
ROWS_AGG = EP // NS // CH
NPT = NP // NS

_f32 = jnp.float32
_i32 = jnp.int32


def _elu(x):
    return jnp.where(x > 0, x, jnp.exp(jnp.minimum(x, 0.0)) - 1.0)


def _tc1_body(feat_ref, w1_ref, att_ref, ab_ref):
    w1a = jnp.dot(w1_ref[...], att_ref[...], preferred_element_type=_f32)
    ab_ref[...] = jnp.dot(feat_ref[...], w1a, preferred_element_type=_f32)


def _tc1(featp, w1, att2):
    R = 1024
    grid = (NP // R,)
    return pl.pallas_call(
        _tc1_body,
        grid=grid,
        in_specs=[
            pl.BlockSpec((R, IN_DIM), lambda i: (i, 0)),
            pl.BlockSpec((IN_DIM, HID), lambda i: (0, 0)),
            pl.BlockSpec((HID, NC), lambda i: (0, 0)),
        ],
        out_specs=pl.BlockSpec((R, NC), lambda i: (i, 0)),
        out_shape=jax.ShapeDtypeStruct((NP, NC), _f32),
    )(featp, w1, att2)


def _tc2_body(dp_ref, out_ref):
    out_ref[...] = 1.0 / (dp_ref[0] + dp_ref[1] + 1e-16)


def _tc2(denom_parts3):
    return pl.pallas_call(
        _tc2_body,
        out_shape=jax.ShapeDtypeStruct((NP // 128, 128), _f32),
    )(denom_parts3)


def _tc3_body(af_ref, w1_ref, w2_ref, h2_ref):
    h1 = _elu(jnp.dot(af_ref[...], w1_ref[...], preferred_element_type=_f32))
    h2_ref[...] = jnp.dot(h1, w2_ref[...], preferred_element_type=_f32)


def _tc3(af, w1, w2):
    R = 512
    grid = (NP // R,)
    return pl.pallas_call(
        _tc3_body,
        grid=grid,
        in_specs=[
            pl.BlockSpec((R, IN_DIM), lambda i: (i, 0)),
            pl.BlockSpec((IN_DIM, HID), lambda i: (0, 0)),
            pl.BlockSpec((HID, OUT_DIM), lambda i: (0, 0)),
        ],
        out_specs=pl.BlockSpec((R, OUT_DIM), lambda i: (i, 0)),
        out_shape=jax.ShapeDtypeStruct((NP, OUT_DIM), _f32),
    )(af, w1, w2)


def _tc4_body(ah_ref, w2_ref, w1_ref, h4_ref, pi_ref, disp_ref, mean_ref):
    ah = ah_ref[0] + ah_ref[1]
    t = lax.dot_general(ah, w2_ref[...], (((1,), (1,)), ((), ())),
                        preferred_element_type=_f32)
    h3 = _elu(t)
    h4 = lax.dot_general(h3, w1_ref[...], (((1,), (1,)), ((), ())),
                         preferred_element_type=_f32)
    h4_ref[...] = h4
    pi_ref[...] = 1.0 / (1.0 + jnp.exp(-h4))
    sp = jnp.log(1.0 + jnp.exp(-jnp.abs(h4))) + jnp.maximum(h4, 0.0)
    disp_ref[...] = jnp.clip(sp, 0.0001, 10000.0)
    mean_ref[...] = jnp.clip(jnp.exp(h4), 1e-05, 1000000.0)


def _tc4(ah_parts, w2, w1):
    R = 1000
    grid = (N // R,)
    o = jax.ShapeDtypeStruct((N, IN_DIM), _f32)
    return pl.pallas_call(
        _tc4_body,
        grid=grid,
        in_specs=[
            pl.BlockSpec((NC, R, OUT_DIM), lambda i: (0, i, 0)),
            pl.BlockSpec((HID, OUT_DIM), lambda i: (0, 0)),
            pl.BlockSpec((IN_DIM, HID), lambda i: (0, 0)),
        ],
        out_specs=[pl.BlockSpec((R, IN_DIM), lambda i: (i, 0))] * 4,
        out_shape=[o, o, o, o],
    )(ah_parts, w2, w1)


@functools.lru_cache(maxsize=None)
def _mesh():
    return plsc.VectorSubcoreMesh(core_axis_name="c", subcore_axis_name="s",
                                  num_cores=NC, num_subcores=NS)


def _zero_vmem(ref, nrows):
    z = jnp.zeros((L,), _f32)

    def body(j, _):
        for q in range(CH // L):
            ref[j, pl.ds(q * L, L)] = z
        return 0

    lax.fori_loop(0, nrows, body, 0)


def _b1_body(src_hbm, dst_hbm, asrc_hbm, adst_hbm, ex_hbm, dpart_hbm,
             asv, adv, srcv, dstv, exv, zbuf, denom_sh):
    c = lax.axis_index("c")
    s = lax.axis_index("s")
    wid = s * NC + c

    pltpu.sync_copy(asrc_hbm, asv)
    pltpu.sync_copy(adst_hbm, adv)
    pltpu.sync_copy(src_hbm.at[pl.ds(wid * ROWS_B1, ROWS_B1)], srcv)
    pltpu.sync_copy(dst_hbm.at[pl.ds(wid * ROWS_B1, ROWS_B1)], dstv)

    z = jnp.zeros((L,), _f32)
    def zb(j, _):
        zbuf[pl.ds(j * L, L)] = z
        return 0
    lax.fori_loop(0, NPT // L, zb, 0)
    pltpu.sync_copy(zbuf, denom_sh.at[pl.ds(s * NPT, NPT)])
    plsc.subcore_barrier()

    def chunk(j, _):
        for q in range(CH // L):
            sv = srcv[j, pl.ds(q * L, L)]
            dv = dstv[j, pl.ds(q * L, L)]
            a1 = plsc.load_gather(asv, [sv])
            a2 = plsc.load_gather(adv, [dv])
            x = a1 + a2
            sig = 1.0 / (1.0 + jnp.exp(-x))
            exv[j, pl.ds(q * L, L)] = jnp.exp(sig)
        return 0

    lax.fori_loop(0, ROWS_B1, chunk, 0)

    def scat(j, _):
        pltpu.sync_copy(exv.at[j], denom_sh.at[dstv.at[j]], add=True)
        return 0

    lax.fori_loop(0, ROWS_B1, scat, 0)
    pltpu.sync_copy(exv, ex_hbm.at[pl.ds(wid * ROWS_B1, ROWS_B1)])
    plsc.subcore_barrier()

    pltpu.sync_copy(denom_sh.at[pl.ds(s * NPT, NPT)],
                    dpart_hbm.at[pl.ds(c * NP + s * NPT, NPT)])


@functools.lru_cache(maxsize=None)
def _b1_kernel():
    return pl.kernel(
        _b1_body,
        out_type=[
            jax.ShapeDtypeStruct((EP // CH, CH), _f32),
            jax.ShapeDtypeStruct((NC * NP,), _f32),
        ],
        mesh=_mesh(),
        compiler_params=pltpu.CompilerParams(use_tc_tiling_on_sc=False, needs_layout_passes=False),
        scratch_types=[
            pltpu.VMEM((NP,), _f32),
            pltpu.VMEM((NP,), _f32),
            pltpu.VMEM((ROWS_B1, CH), _i32),
            pltpu.VMEM((ROWS_B1, CH), _i32),
            pltpu.VMEM((ROWS_B1, CH), _f32),
            pltpu.VMEM((NPT,), _f32),
            pltpu.VMEM_SHARED((NP,), _f32),
        ],
    )


G_AGG = 40


def _make_agg_body(d, chunks_per_tile, dim_split):
    n_grp = chunks_per_tile // G_AGG
    qd = d // L

    def body(src_hbm, dst_hbm, ex_hbm, dpart_hbm, table_hbm, out_hbm,
             srcv, dstv, exv, invdv, wv, rows0, rows1,
             gsem0, gsem1, ssem0, ssem1, accum_sh):
        c = lax.axis_index("c")
        s = lax.axis_index("s")

        pltpu.sync_copy(dpart_hbm, invdv)

        if dim_split:
            off = (c * NP).astype(_i32)
            tile_chunk0 = s * chunks_per_tile
        else:
            off = None
            tile_chunk0 = (c * NS + s) * chunks_per_tile

        z = jnp.zeros((L,), _f32)

        def zrow(e, _):
            for q in range(qd):
                rows0[e, pl.ds(q * L, L)] = z
            return 0
        lax.fori_loop(0, CH, zrow, 0)
        for r0 in range(0, NPT, CH):
            pltpu.sync_copy(rows0, accum_sh.at[pl.ds(s * NPT + r0, CH)])
        plsc.subcore_barrier()

        bufs = ((rows0, gsem0, ssem0), (rows1, gsem1, ssem1))

        def group(g, _):
            base = tile_chunk0 + g * G_AGG
            pltpu.sync_copy(src_hbm.at[pl.ds(base, G_AGG)], srcv)
            pltpu.sync_copy(dst_hbm.at[pl.ds(base, G_AGG)], dstv)
            pltpu.sync_copy(ex_hbm.at[pl.ds(base, G_AGG)], exv)

            for j in range(G_AGG):
                for q in range(CH // L):
                    dv = dstv[j, pl.ds(q * L, L)]
                    iv = plsc.load_gather(invdv, [dv])
                    wv[j, pl.ds(q * L, L)] = exv[j, pl.ds(q * L, L)] * iv
                    if off is not None:
                        srcv[j, pl.ds(q * L, L)] = (
                            srcv[j, pl.ds(q * L, L)] + off)

            gd = [None] * G_AGG
            sd = [None] * G_AGG
            gd[0] = pltpu.async_copy(table_hbm.at[srcv.at[0]], rows0, gsem0)
            for j in range(G_AGG):
                rb, _, ssb = bufs[j & 1]
                ro, gso, _ = bufs[(j + 1) & 1]
                if j + 1 < G_AGG:
                    if j >= 1:
                        sd[j - 1].wait()
                    gd[j + 1] = pltpu.async_copy(
                        table_hbm.at[srcv.at[j + 1]], ro, gso)
                gd[j].wait()

                @pl.loop(0, CH, unroll=4)
                def _scale(e):
                    wbc = plsc.load_gather(
                        wv, [jnp.full((L,), j, _i32), jnp.full((L,), e, _i32)])
                    for q in range(qd):
                        rb[e, pl.ds(q * L, L)] = rb[e, pl.ds(q * L, L)] * wbc

                sd[j] = pltpu.async_copy(rb, accum_sh.at[dstv.at[j]], ssb,
                                         add=True)
            sd[G_AGG - 2].wait()
            sd[G_AGG - 1].wait()
            return 0

        lax.fori_loop(0, n_grp, group, 0)
        plsc.subcore_barrier()
        pltpu.sync_copy(accum_sh.at[pl.ds(s * NPT, NPT)],
                        out_hbm.at[pl.ds(c * NP + s * NPT, NPT)])

    return body


@functools.lru_cache(maxsize=None)
def _agg_kernel(d, chunks_per_tile, dim_split):
    return pl.kernel(
        _make_agg_body(d, chunks_per_tile, dim_split),
        out_type=jax.ShapeDtypeStruct((NC * NP, d), _f32),
        mesh=_mesh(),
        compiler_params=pltpu.CompilerParams(use_tc_tiling_on_sc=False, needs_layout_passes=False),
        scratch_types=[
            pltpu.VMEM((G_AGG, CH), _i32),
            pltpu.VMEM((G_AGG, CH), _i32),
            pltpu.VMEM((G_AGG, CH), _f32),
            pltpu.VMEM((NP,), _f32),
            pltpu.VMEM((G_AGG, CH), _f32),
            pltpu.VMEM((CH, d), _f32),
            pltpu.VMEM((CH, d), _f32),
            pltpu.SemaphoreType.DMA,
            pltpu.SemaphoreType.DMA,
            pltpu.SemaphoreType.DMA,
            pltpu.SemaphoreType.DMA,
            pltpu.VMEM_SHARED((NP, d), _f32),
        ],
    )


def _csl_body(gat_hbm, sct_hbm, h2_hbm, out_hbm,
              gv, sv, onesv, rows0, rows1, cntv, mv,
              gsem0, gsem1, ssem0, ssem1, csem, accum_sh, cnt_sh):
    c = lax.axis_index("c")
    s = lax.axis_index("s")
    base = c * (EP // CH) + s * ROWS_CSL

    pltpu.sync_copy(gat_hbm.at[pl.ds(base, ROWS_CSL)], gv)
    pltpu.sync_copy(sct_hbm.at[pl.ds(base, ROWS_CSL)], sv)

    one = jnp.full((L,), 1.0, _f32)
    z = jnp.zeros((L,), _f32)
    for q in range(CH // L):
        onesv[pl.ds(q * L, L)] = one

    def zrow(e, _):
        for q in range(OUT_DIM // L):
            rows0[e, pl.ds(q * L, L)] = z
        return 0
    lax.fori_loop(0, CH, zrow, 0)

    for r0 in range(0, NPT, CH):
        pltpu.sync_copy(rows0, accum_sh.at[pl.ds(s * NPT + r0, CH)])
    def zb(k, _):
        cntv[pl.ds(k * L, L)] = z
        return 0
    lax.fori_loop(0, NPT // L, zb, 0)
    pltpu.sync_copy(cntv, cnt_sh.at[pl.ds(s * NPT, NPT)])
    plsc.subcore_barrier()

    bufs = ((rows0, gsem0, ssem0), (rows1, gsem1, ssem1))
    gd = [None] * ROWS_CSL
    sd = [None] * ROWS_CSL
    cd = [None] * ROWS_CSL
    gd[0] = pltpu.async_copy(h2_hbm.at[gv.at[0]], rows0, gsem0)
    for j in range(ROWS_CSL):
        rb, _, ssb = bufs[j & 1]
        ro, gso, _ = bufs[(j + 1) & 1]
        if j + 1 < ROWS_CSL:
            if j >= 1:
                sd[j - 1].wait()
            gd[j + 1] = pltpu.async_copy(h2_hbm.at[gv.at[j + 1]], ro, gso)
        gd[j].wait()
        sd[j] = pltpu.async_copy(rb, accum_sh.at[sv.at[j]], ssb, add=True)
        cd[j] = pltpu.async_copy(onesv, cnt_sh.at[sv.at[j]], csem, add=True)
        if j >= 1:
            cd[j - 1].wait()
    sd[ROWS_CSL - 2].wait()
    sd[ROWS_CSL - 1].wait()
    cd[ROWS_CSL - 1].wait()
    plsc.subcore_barrier()

    pltpu.sync_copy(cnt_sh.at[pl.ds(s * NPT, NPT)], cntv)
    pltpu.sync_copy(accum_sh.at[pl.ds(s * NPT, NPT)], mv)

    def div(r, _):
        cb = plsc.load_gather(cntv, [jnp.full((L,), r, _i32)])
        rec = 1.0 / jnp.maximum(cb, 1.0)
        for q in range(OUT_DIM // L):
            mv[r, pl.ds(q * L, L)] = mv[r, pl.ds(q * L, L)] * rec
        return 0

    lax.fori_loop(0, NPT, div, 0)
    pltpu.sync_copy(mv, out_hbm.at[pl.ds(c * NP + s * NPT, NPT)])


ROWS_CSL = EP // NS // CH


@functools.lru_cache(maxsize=None)
def _csl_kernel():
    return pl.kernel(
        _csl_body,
        out_type=jax.ShapeDtypeStruct((NC * NP, OUT_DIM), _f32),
        mesh=_mesh(),
        compiler_params=pltpu.CompilerParams(use_tc_tiling_on_sc=False, needs_layout_passes=False),
        scratch_types=[
            pltpu.VMEM((ROWS_CSL, CH), _i32),
            pltpu.VMEM((ROWS_CSL, CH), _i32),
            pltpu.VMEM((CH,), _f32),
            pltpu.VMEM((CH, OUT_DIM), _f32),
            pltpu.VMEM((CH, OUT_DIM), _f32),
            pltpu.VMEM((NPT,), _f32),
            pltpu.VMEM((NPT, OUT_DIM), _f32),
            pltpu.SemaphoreType.DMA,
            pltpu.SemaphoreType.DMA,
            pltpu.SemaphoreType.DMA,
            pltpu.SemaphoreType.DMA,
            pltpu.SemaphoreType.DMA,
            pltpu.VMEM_SHARED((NP, OUT_DIM), _f32),
            pltpu.VMEM_SHARED((NP,), _f32),
        ],
    )


def _pad_edges(idx):
    return jnp.pad(idx, (0, EP - E), constant_values=N).reshape(EP // CH, CH)


def kernel(features, edge_index, CL_graph, W1, W2, att_src1, att_dst1):
    featp = jnp.pad(features, ((0, NP - N), (0, 0)))
    src2d = _pad_edges(edge_index[0].astype(_i32))
    dst2d = _pad_edges(edge_index[1].astype(_i32))

    ab = _tc1(featp, W1, jnp.stack([att_src1, att_dst1], axis=1))

    ex2d, dparts = _b1_kernel()(src2d, dst2d, ab[:, 0], ab[:, 1])
    invd = _tc2(dparts.reshape(NC, NP // 128, 128)).reshape(NP)

    featsplit = jnp.concatenate([featp[:, :64], featp[:, 64:]], axis=0)
    acc1 = _agg_kernel(64, EP // CH // NS, True)(
        src2d, dst2d, ex2d, invd, featsplit)
    af = jnp.concatenate([acc1[:NP], acc1[NP:]], axis=1)
    h2p = _tc3(af, W1, W2)

    acc3 = _agg_kernel(OUT_DIM, EP // CH // (NC * NS), False)(
        src2d, dst2d, ex2d, invd, h2p)
    ah_parts = acc3.reshape(NC, NP, OUT_DIM)[:, :N]

    h4, pi, disp, mean = _tc4(ah_parts, W2, W1)

    gat = jnp.concatenate([_pad_edges(edge_index[1].astype(_i32)),
                           _pad_edges(CL_graph[1].astype(_i32))], axis=0)
    sct = jnp.concatenate([_pad_edges(edge_index[0].astype(_i32)),
                           _pad_edges(CL_graph[0].astype(_i32))], axis=0)
    csl = _csl_kernel()(gat, sct, h2p)

    return (h2p[:N], csl[:N], csl[NP:NP + N], mean, disp, pi, h4)
